# baseline (speedup 1.0000x reference)
import jax
import jax.numpy as jnp
from jax import lax
from jax.experimental import pallas as pl
from jax.experimental.pallas import tpu as pltpu
from jax.experimental.pallas import tpu_sc as plsc

START_VAL = 1.0
STOP_VAL = 2.0

B = 1024
L = 4096
LP1 = L + 1

_info = plsc.get_sparse_core_info()
NC, NS = _info.num_cores, _info.num_subcores
NW = NC * NS
ROWS_PER_W = B // NW
CHUNK = 8
N_CHUNKS = ROWS_PER_W // CHUNK
SLICES = CHUNK * L // 16


def _collate_body(samples, inp_out, len1_out, tgt_out, len2_out,
                  stage, img, len_buf, sem_in, sem_out):
    wid = lax.axis_index("s") * NC + lax.axis_index("c")
    base = wid * ROWS_PER_W

    iota = lax.iota(jnp.int32, 16)

    lenv = jnp.full((16,), LP1, dtype=jnp.int32)
    for j in range(ROWS_PER_W // 16):
        len_buf[pl.ds(j * 16, 16)] = lenv
    pltpu.sync_copy(len_buf, len1_out.at[pl.ds(base, ROWS_PER_W)])
    pltpu.sync_copy(len_buf, len2_out.at[pl.ds(base, ROWS_PER_W)])

    def do_chunk(rr, ch, shift, out_ref):
        cps = [pltpu.async_copy(samples.at[rr + r, ch],
                                stage.at[pl.ds(r * L, L)], sem_in)
               for r in range(CHUNK)]
        for cp in cps:
            cp.wait()

        def cp_step(i, _):
            v = stage[pl.ds(i * 16, 16)]
            img[pl.ds(i * 16 + lax.shift_right_logical(i, 8) + shift, 16)] = v
            return ()
        lax.fori_loop(0, SLICES, cp_step, ())

        for r in range(CHUNK):
            if shift:
                off = r * LP1
                va = img[pl.ds(off, 16)]
                img[pl.ds(off, 16)] = jnp.where(iota == 0, START_VAL, va)
            else:
                off = r * LP1 + LP1 - 16
                vb = img[pl.ds(off, 16)]
                img[pl.ds(off, 16)] = jnp.where(iota == 15, STOP_VAL, vb)

        off = pl.multiple_of(rr * LP1, 8)
        pltpu.async_copy(img, out_ref.at[pl.ds(off, CHUNK * LP1)],
                         sem_out).wait()

    for c in range(N_CHUNKS):
        rr = base + c * CHUNK
        do_chunk(rr, 0, 1, inp_out)
        do_chunk(rr, 1, 0, tgt_out)


@jax.jit
def _collate(samples):
    mesh = plsc.VectorSubcoreMesh(core_axis_name="c", subcore_axis_name="s")
    f = pl.kernel(
        _collate_body,
        out_type=(
            jax.ShapeDtypeStruct((B * LP1,), jnp.float32),
            jax.ShapeDtypeStruct((B,), jnp.int32),
            jax.ShapeDtypeStruct((B * LP1,), jnp.float32),
            jax.ShapeDtypeStruct((B,), jnp.int32),
        ),
        mesh=mesh,
        scratch_types=[
            pltpu.VMEM((CHUNK * L,), jnp.float32),
            pltpu.VMEM((CHUNK * LP1,), jnp.float32),
            pltpu.VMEM((ROWS_PER_W,), jnp.int32),
            pltpu.SemaphoreType.DMA,
            pltpu.SemaphoreType.DMA,
        ],
    )
    inp_flat, len1, tgt_flat, len2 = f(samples)
    return (inp_flat.reshape(B, LP1), len1, tgt_flat.reshape(B, LP1), len2)


def kernel(samples):
    return _collate(samples)
